# baseline (speedup 1.0000x reference)
import functools

import jax
import jax.numpy as jnp
from jax import lax
from jax.experimental import pallas as pl
from jax.experimental.pallas import tpu as pltpu
from jax.experimental.pallas import tpu_sc as plsc

NIMG = 255
H = 128
W = 128
HW = H * W
NCELLS = NIMG * HW
LAM_NOOBJ = 0.5

NLAB = 100_000
NPAD = 100_352
WIN = 2048
NWIN = NPAD // WIN
CHUNK = 4 * HW
NW = 32
SUB = 8192
BLK = 128
LANES = 16


def _dense_obj_sq(output):
    def body(x_ref, o_ref):
        i = pl.program_id(0)

        @pl.when(i == 0)
        def _():
            o_ref[...] = jnp.zeros_like(o_ref)

        v = x_ref[...]
        o_ref[...] += jnp.sum(v * v)

    out = pl.pallas_call(
        body,
        grid=(NIMG,),
        in_specs=[pl.BlockSpec((1, 1, H, W), lambda i: (i, 4, 0, 0))],
        out_specs=pl.BlockSpec((8, 128), lambda i: (0, 0)),
        out_shape=jax.ShapeDtypeStruct((8, 128), jnp.float32),
    )(output)
    return out[0, 0]


def _sc_partials_kernel():
    mesh = plsc.VectorSubcoreMesh(core_axis_name="c", subcore_axis_name="s")
    scratch = [
        pltpu.VMEM((CHUNK,), jnp.int32),
        pltpu.VMEM((WIN, 16), jnp.int32),
        pltpu.VMEM((SUB,), jnp.int32),
        pltpu.VMEM((SUB,), jnp.int32),
        pltpu.VMEM((4 * LANES,), jnp.float32),
        pltpu.VMEM((64,), jnp.float32),
        pltpu.VMEM((BLK,), jnp.int32),
        pltpu.VMEM((BLK,), jnp.int32),
        pltpu.VMEM((BLK, 16), jnp.int32),
        pltpu.VMEM((BLK, 8), jnp.float32),
        pltpu.SemaphoreType.DMA,
    ]

    @functools.partial(
        pl.kernel,
        out_type=jax.ShapeDtypeStruct((NW, 64), jnp.float32),
        mesh=mesh,
        scratch_types=scratch,
        compiler_params=pltpu.CompilerParams(needs_layout_passes=False, use_tc_tiling_on_sc=False),
    )
    def k(labels_hbm, outT_hbm, out_hbm, wbuf, win, lst_r, lst_f, accs,
          res, ixr, ixf, dlab, dout, sem):
        wid = lax.axis_index("s") * 2 + lax.axis_index("c")
        lanes = lax.iota(jnp.int32, 16)

        for q in range(4):
            accs[pl.ds(q * LANES, LANES)] = jnp.zeros((LANES,), jnp.float32)

        def chunk_body(t, _):
            cid = wid + t * NW
            lo = cid * CHUNK
            clen = jnp.minimum(NCELLS - lo, CHUNK)
            hi = lo + clen

            def zb(i, _):
                z = jnp.zeros((LANES,), jnp.int32)
                for u in range(4):
                    off = pl.multiple_of(i * (4 * LANES) + u * LANES, LANES)
                    wbuf[pl.ds(off, LANES)] = z
                return 0

            lax.fori_loop(0, CHUNK // (4 * LANES), zb, 0)

            def win_body(w, _):
                off = pl.multiple_of(w * WIN, 8)
                pltpu.sync_copy(labels_hbm.at[pl.ds(off, WIN)], win)

                def vec_body(j, _):
                    rows_l = j * LANES + lanes
                    x = plsc.load_gather(win, [rows_l, lanes * 0])
                    y = plsc.load_gather(win, [rows_l, lanes * 0 + 1])
                    img = plsc.load_gather(win, [rows_l, lanes * 0 + 9])
                    f = (img - 1) * HW + (y >> 1) * W + (x >> 1)
                    m = (img >= 1) & (f >= lo) & (f < hi)
                    fl = jnp.where(m, f - lo, 0)
                    rowp1 = w * WIN + j * LANES + lanes + 1
                    g = plsc.load_gather(wbuf, [fl], mask=m)
                    need = m & (rowp1 > jnp.where(m, g, rowp1))

                    def cond(nd):
                        return jnp.any(nd)

                    def body(nd):
                        plsc.store_scatter(wbuf, [fl], rowp1, mask=nd)
                        g2 = plsc.load_gather(wbuf, [fl], mask=m)
                        return m & (rowp1 > jnp.where(m, g2, rowp1))

                    lax.while_loop(cond, body, need)
                    return 0

                lax.fori_loop(0, WIN // LANES, vec_body, 0)
                return 0

            lax.fori_loop(0, NWIN, win_body, 0)

            def sub_body(si, _):
                sub_base = si * SUB

                def ex_body(j, cnt):
                    off = pl.multiple_of(sub_base + j * LANES, LANES)
                    wv = wbuf[pl.ds(off, LANES)]
                    m = wv > 0
                    ones = jnp.where(m, 1, 0)
                    pos = plsc.cumsum(ones) - 1 + cnt
                    posc = jnp.where(m, pos, 0)
                    fvec = lo + off + lanes
                    plsc.store_scatter(lst_r, [posc], wv - 1, mask=m)
                    plsc.store_scatter(lst_f, [posc], fvec, mask=m)
                    return cnt + jnp.sum(ones)

                cnt = lax.fori_loop(0, SUB // LANES, ex_body, 0)
                nblk = (cnt + (BLK - 1)) // BLK

                def blk_body(b, _):
                    for kk in range(BLK // LANES):
                        off = pl.multiple_of(b * BLK + kk * LANES, LANES)
                        lid = b * BLK + kk * LANES + lanes
                        ok = lid < cnt
                        so = pl.ds(kk * LANES, LANES)
                        ixr[so] = jnp.where(ok, lst_r[pl.ds(off, LANES)], 0)
                        ixf[so] = jnp.where(ok, lst_f[pl.ds(off, LANES)], 0)
                    d1 = pltpu.async_copy(outT_hbm.at[ixf], dout, sem)
                    d2 = pltpu.async_copy(labels_hbm.at[ixr], dlab, sem)
                    d1.wait()
                    d2.wait()
                    for kk in range(BLK // LANES):
                        rows = kk * LANES + lanes
                        so = pl.ds(kk * LANES, LANES)
                        lid = b * BLK + kk * LANES + lanes
                        okf = jnp.where(lid < cnt, 1.0, 0.0).astype(jnp.float32)
                        po = plsc.load_gather(dout, [rows, lanes * 0 + 4])
                        dd = []
                        for c in range(4):
                            p = plsc.load_gather(dout, [rows, lanes * 0 + c])
                            v = plsc.load_gather(
                                dlab, [rows, lanes * 0 + 4 + c]
                            ).astype(jnp.float32)
                            dd.append(p - v)
                        accs[pl.ds(0, LANES)] += okf * po * po
                        accs[pl.ds(LANES, LANES)] += okf * (
                            dd[0] * dd[0] + dd[1] * dd[1])
                        accs[pl.ds(2 * LANES, LANES)] += okf * (
                            dd[2] * dd[2] + dd[3] * dd[3])
                        accs[pl.ds(3 * LANES, LANES)] += okf
                    return 0

                lax.fori_loop(0, nblk, blk_body, 0)
                return 0

            lax.fori_loop(0, clen // SUB, sub_body, 0)
            return 0

        lax.fori_loop(0, 2, chunk_body, 0)

        for q in range(4):
            res[pl.ds(q * LANES, LANES)] = accs[pl.ds(q * LANES, LANES)]
        pltpu.sync_copy(res, out_hbm.at[wid])

    return k


_SC_KERNEL = _sc_partials_kernel()


def kernel(output, train_labels):
    s_all = _dense_obj_sq(output)

    labels2d = jnp.pad(train_labels, ((0, NPAD - NLAB), (0, 6)))
    outT = jnp.pad(output.transpose(0, 2, 3, 1),
                   ((0, 0), (0, 0), (0, 0), (0, 3))).reshape(NCELLS, 8)
    partials = _SC_KERNEL(labels2d, outT)

    sums = partials.reshape(NW, 4, LANES).sum(axis=(0, 2))
    s_obj, s_vel, s_acc, cntf = sums[0], sums[1], sums[2], sums[3]
    noobj_cnt = NCELLS - cntf
    noobj = (s_all - s_obj) / jnp.maximum(noobj_cnt, 1.0)
    vel = s_vel / jnp.maximum(2.0 * cntf, 1.0)
    acc = s_acc / jnp.maximum(2.0 * cntf, 1.0)
    return LAM_NOOBJ * noobj + vel + acc


# speedup vs baseline: 3.4561x; 3.4561x over previous
import functools

import jax
import jax.numpy as jnp
from jax import lax
from jax.experimental import pallas as pl
from jax.experimental.pallas import tpu as pltpu
from jax.experimental.pallas import tpu_sc as plsc

NIMG = 255
H = 128
W = 128
HW = H * W
NCELLS = NIMG * HW
LAM_NOOBJ = 0.5

NLAB = 100_000
NPAD = 100_352
WIN = 2048
NWIN = NPAD // WIN
CHUNK = 4 * HW
NW = 32
SUB = 8192
BLK = 128
LANES = 16


def _dense_obj_sq(output):
    def body(x_ref, o_ref):
        i = pl.program_id(0)

        @pl.when(i == 0)
        def _():
            o_ref[...] = jnp.zeros_like(o_ref)

        v = x_ref[...]
        o_ref[...] += jnp.sum(v * v)

    out = pl.pallas_call(
        body,
        grid=(NIMG,),
        in_specs=[pl.BlockSpec((1, 1, H, W), lambda i: (i, 4, 0, 0))],
        out_specs=pl.BlockSpec((8, 128), lambda i: (0, 0)),
        out_shape=jax.ShapeDtypeStruct((8, 128), jnp.float32),
    )(output)
    return out[0, 0]


def _sc_partials_kernel():
    mesh = plsc.VectorSubcoreMesh(core_axis_name="c", subcore_axis_name="s")
    scratch = [
        pltpu.VMEM((CHUNK,), jnp.int32),
        pltpu.VMEM((WIN, 16), jnp.int32),
        pltpu.VMEM((SUB,), jnp.int32),
        pltpu.VMEM((SUB,), jnp.int32),
        pltpu.VMEM((4 * LANES,), jnp.float32),
        pltpu.VMEM((64,), jnp.float32),
        pltpu.VMEM((BLK,), jnp.int32),
        pltpu.VMEM((5 * BLK,), jnp.int32),
        pltpu.VMEM((BLK, 16), jnp.int32),
        pltpu.VMEM((5 * BLK,), jnp.float32),
        pltpu.SemaphoreType.DMA,
    ]

    @functools.partial(
        pl.kernel,
        out_type=jax.ShapeDtypeStruct((NW, 64), jnp.float32),
        mesh=mesh,
        scratch_types=scratch,
        compiler_params=pltpu.CompilerParams(needs_layout_passes=False, use_tc_tiling_on_sc=False),
    )
    def k(labels_hbm, outT_hbm, out_hbm, wbuf, win, lst_r, lst_f, accs,
          res, ixr, ixf, dlab, dout, sem):
        wid = lax.axis_index("s") * 2 + lax.axis_index("c")
        lanes = lax.iota(jnp.int32, 16)

        for q in range(4):
            accs[pl.ds(q * LANES, LANES)] = jnp.zeros((LANES,), jnp.float32)

        def chunk_body(t, _):
            cid = wid + t * NW
            lo = cid * CHUNK
            clen = jnp.minimum(NCELLS - lo, CHUNK)
            hi = lo + clen

            def zb(i, _):
                z = jnp.zeros((LANES,), jnp.int32)
                for u in range(4):
                    off = pl.multiple_of(i * (4 * LANES) + u * LANES, LANES)
                    wbuf[pl.ds(off, LANES)] = z
                return 0

            lax.fori_loop(0, CHUNK // (4 * LANES), zb, 0)

            def win_body(w, _):
                off = pl.multiple_of(w * WIN, 8)
                pltpu.sync_copy(labels_hbm.at[pl.ds(off, WIN)], win)

                def vec_body(j, _):
                    rows_l = j * LANES + lanes
                    x = plsc.load_gather(win, [rows_l, lanes * 0])
                    y = plsc.load_gather(win, [rows_l, lanes * 0 + 1])
                    img = plsc.load_gather(win, [rows_l, lanes * 0 + 9])
                    f = (img - 1) * HW + (y >> 1) * W + (x >> 1)
                    m = (img >= 1) & (f >= lo) & (f < hi)
                    fl = jnp.where(m, f - lo, 0)
                    rowp1 = w * WIN + j * LANES + lanes + 1
                    g = plsc.load_gather(wbuf, [fl], mask=m)
                    need = m & (rowp1 > jnp.where(m, g, rowp1))

                    def cond(nd):
                        return jnp.any(nd)

                    def body(nd):
                        plsc.store_scatter(wbuf, [fl], rowp1, mask=nd)
                        g2 = plsc.load_gather(wbuf, [fl], mask=m)
                        return m & (rowp1 > jnp.where(m, g2, rowp1))

                    lax.while_loop(cond, body, need)
                    return 0

                lax.fori_loop(0, WIN // LANES, vec_body, 0)
                return 0

            lax.fori_loop(0, NWIN, win_body, 0)

            def sub_body(si, _):
                sub_base = si * SUB

                def ex_body(j, cnt):
                    off = pl.multiple_of(sub_base + j * LANES, LANES)
                    wv = wbuf[pl.ds(off, LANES)]
                    m = wv > 0
                    ones = jnp.where(m, 1, 0)
                    pos = plsc.cumsum(ones) - 1 + cnt
                    posc = jnp.where(m, pos, 0)
                    fvec = lo + off + lanes
                    plsc.store_scatter(lst_r, [posc], wv - 1, mask=m)
                    plsc.store_scatter(lst_f, [posc], fvec, mask=m)
                    return cnt + jnp.sum(ones)

                cnt = lax.fori_loop(0, SUB // LANES, ex_body, 0)
                nblk = (cnt + (BLK - 1)) // BLK

                def blk_body(b, _):
                    for kk in range(BLK // LANES):
                        off = pl.multiple_of(b * BLK + kk * LANES, LANES)
                        lid = b * BLK + kk * LANES + lanes
                        ok = lid < cnt
                        so = pl.ds(kk * LANES, LANES)
                        ixr[so] = jnp.where(ok, lst_r[pl.ds(off, LANES)], 0)
                        fv = jnp.where(ok, lst_f[pl.ds(off, LANES)], 0)
                        e0 = fv + (fv >> 14) * (4 * HW)
                        for q in range(5):
                            ixf[pl.ds(q * BLK + kk * LANES, LANES)] = e0 + q * HW
                    d1 = pltpu.async_copy(outT_hbm.at[ixf], dout, sem)
                    d2 = pltpu.async_copy(labels_hbm.at[ixr], dlab, sem)
                    d1.wait()
                    d2.wait()
                    for kk in range(BLK // LANES):
                        rows = kk * LANES + lanes
                        so = pl.ds(kk * LANES, LANES)
                        lid = b * BLK + kk * LANES + lanes
                        okf = jnp.where(lid < cnt, 1.0, 0.0).astype(jnp.float32)
                        po = dout[pl.ds(4 * BLK + kk * LANES, LANES)]
                        dd = []
                        for c in range(4):
                            p = dout[pl.ds(c * BLK + kk * LANES, LANES)]
                            v = plsc.load_gather(
                                dlab, [rows, lanes * 0 + 4 + c]
                            ).astype(jnp.float32)
                            dd.append(p - v)
                        accs[pl.ds(0, LANES)] += okf * po * po
                        accs[pl.ds(LANES, LANES)] += okf * (
                            dd[0] * dd[0] + dd[1] * dd[1])
                        accs[pl.ds(2 * LANES, LANES)] += okf * (
                            dd[2] * dd[2] + dd[3] * dd[3])
                        accs[pl.ds(3 * LANES, LANES)] += okf
                    return 0

                lax.fori_loop(0, nblk, blk_body, 0)
                return 0

            lax.fori_loop(0, clen // SUB, sub_body, 0)
            return 0

        lax.fori_loop(0, 2, chunk_body, 0)

        for q in range(4):
            res[pl.ds(q * LANES, LANES)] = accs[pl.ds(q * LANES, LANES)]
        pltpu.sync_copy(res, out_hbm.at[wid])

    return k


_SC_KERNEL = _sc_partials_kernel()


def kernel(output, train_labels):
    s_all = _dense_obj_sq(output)

    labels2d = jnp.pad(train_labels, ((0, NPAD - NLAB), (0, 6)))
    outT = output.reshape(-1)
    partials = _SC_KERNEL(labels2d, outT)

    sums = partials.reshape(NW, 4, LANES).sum(axis=(0, 2))
    s_obj, s_vel, s_acc, cntf = sums[0], sums[1], sums[2], sums[3]
    noobj_cnt = NCELLS - cntf
    noobj = (s_all - s_obj) / jnp.maximum(noobj_cnt, 1.0)
    vel = s_vel / jnp.maximum(2.0 * cntf, 1.0)
    acc = s_acc / jnp.maximum(2.0 * cntf, 1.0)
    return LAM_NOOBJ * noobj + vel + acc
